# baseline (speedup 1.0000x reference)
import functools

import jax
import jax.numpy as jnp
from jax import lax
from jax.experimental import pallas as pl
from jax.experimental.pallas import tpu as pltpu
from jax.experimental.pallas import tpu_sc as plsc

B, N, D = 2, 512, 128
R_SC = 512
N_TC = N - R_SC
NWB = 16
RW = R_SC // NWB
DV = D // 16


def _tc_body(adj_ref, feats16_ref, feats32_ref, out_ref):
    adj = adj_ref[...]
    feats = feats16_ref[...]
    masked = adj[:, :, None] * feats[None, :, :]
    mins = jnp.min(masked, axis=1)
    maxs = jnp.max(masked, axis=1)
    out_ref[:, 0:D] = feats32_ref[pl.ds(0, N_TC), :]
    out_ref[:, D:2 * D] = mins.astype(jnp.float32)
    out_ref[:, 2 * D:3 * D] = maxs.astype(jnp.float32)


def _tc_call(adj16, feats16, feats):
    return pl.pallas_call(
        _tc_body,
        grid=(B,),
        in_specs=[
            pl.BlockSpec((None, N_TC, N), lambda b: (b, 0, 0)),
            pl.BlockSpec((None, N, D), lambda b: (b, 0, 0)),
            pl.BlockSpec((None, N, D), lambda b: (b, 0, 0)),
        ],
        out_specs=pl.BlockSpec((None, N_TC, 3 * D), lambda b: (b, 0, 0)),
        out_shape=jax.ShapeDtypeStruct((B, N_TC, 3 * D), jnp.float32),
        compiler_params=pltpu.CompilerParams(
            dimension_semantics=("parallel",)),
    )(adj16[:, :N_TC, :], feats16, feats)


@functools.partial(
    pl.kernel,
    out_type=jax.ShapeDtypeStruct((B, R_SC, 3 * D), jnp.float32),
    mesh=plsc.VectorSubcoreMesh(core_axis_name="c", subcore_axis_name="s"),
    scratch_types=[
        pltpu.VMEM((N, D), jnp.float32),
        pltpu.VMEM((RW, N), jnp.float32),
        pltpu.VMEM((RW, 3 * D), jnp.float32),
    ],
)
def _sc_minmax(adj_hbm, feats_hbm, out_hbm, feats_v, adj_v, out_v):
    c = lax.axis_index("c")
    s = lax.axis_index("s")
    wid = s * 2 + c
    b = wid // NWB
    r0 = (N - R_SC) + (wid % NWB) * RW
    o0 = (wid % NWB) * RW
    pltpu.sync_copy(feats_hbm.at[b], feats_v)
    pltpu.sync_copy(adj_hbm.at[b, pl.ds(r0, RW)], adj_v)

    def pair_body(rp, _):
        def j16_body(jv, accs):
            mins0, maxs0, mins1, maxs1 = accs
            a0 = adj_v[2 * rp, pl.ds(jv * 16, 16)]
            a1 = adj_v[2 * rp + 1, pl.ds(jv * 16, 16)]
            mins0, maxs0 = list(mins0), list(maxs0)
            mins1, maxs1 = list(mins1), list(maxs1)
            for k in range(16):
                j = jv * 16 + k
                s0 = a0[k]
                s1 = a1[k]
                for dv in range(DV):
                    f = feats_v[j, pl.ds(dv * 16, 16)]
                    p0 = s0 * f
                    p1 = s1 * f
                    mins0[dv] = jnp.minimum(mins0[dv], p0)
                    maxs0[dv] = jnp.maximum(maxs0[dv], p0)
                    mins1[dv] = jnp.minimum(mins1[dv], p1)
                    maxs1[dv] = jnp.maximum(maxs1[dv], p1)
            return tuple(mins0), tuple(maxs0), tuple(mins1), tuple(maxs1)

        pinf = tuple(jnp.full((16,), jnp.inf, jnp.float32)
                     for _ in range(DV))
        ninf = tuple(jnp.full((16,), -jnp.inf, jnp.float32)
                     for _ in range(DV))
        mins0, maxs0, mins1, maxs1 = lax.fori_loop(
            0, N // 16, j16_body, (pinf, ninf, pinf, ninf))
        for dv in range(DV):
            sl = pl.ds(dv * 16, 16)
            out_v[2 * rp, sl] = feats_v[r0 + 2 * rp, sl]
            out_v[2 * rp + 1, sl] = feats_v[r0 + 2 * rp + 1, sl]
            out_v[2 * rp, pl.ds(D + dv * 16, 16)] = mins0[dv]
            out_v[2 * rp, pl.ds(2 * D + dv * 16, 16)] = maxs0[dv]
            out_v[2 * rp + 1, pl.ds(D + dv * 16, 16)] = mins1[dv]
            out_v[2 * rp + 1, pl.ds(2 * D + dv * 16, 16)] = maxs1[dv]
        return 0

    lax.fori_loop(0, RW // 2, pair_body, 0)
    pltpu.sync_copy(out_v, out_hbm.at[b, pl.ds(o0, RW)])


def kernel(adjMs, feats):
    parts = []
    if N_TC > 0:
        adj16 = adjMs.astype(jnp.bfloat16)
        feats16 = feats.astype(jnp.bfloat16)
        parts.append(_tc_call(adj16, feats16, feats))
    if R_SC > 0:
        parts.append(_sc_minmax(adjMs, feats))
    out = parts[0] if len(parts) == 1 else jnp.concatenate(parts, axis=1)
    return (adjMs, out)


# speedup vs baseline: 1.3560x; 1.3560x over previous
import functools

import jax
import jax.numpy as jnp
from jax import lax
from jax.experimental import pallas as pl
from jax.experimental.pallas import tpu as pltpu
from jax.experimental.pallas import tpu_sc as plsc

B, N, D = 2, 512, 128
R_SC = 512
N_TC = N - R_SC
NWB = 16
RW = R_SC // NWB
DV = D // 16


def _tc_body(adj_ref, feats16_ref, feats32_ref, out_ref):
    adj = adj_ref[...]
    feats = feats16_ref[...]
    masked = adj[:, :, None] * feats[None, :, :]
    mins = jnp.min(masked, axis=1)
    maxs = jnp.max(masked, axis=1)
    out_ref[:, 0:D] = feats32_ref[pl.ds(0, N_TC), :]
    out_ref[:, D:2 * D] = mins.astype(jnp.float32)
    out_ref[:, 2 * D:3 * D] = maxs.astype(jnp.float32)


def _tc_call(adj16, feats16, feats):
    return pl.pallas_call(
        _tc_body,
        grid=(B,),
        in_specs=[
            pl.BlockSpec((None, N_TC, N), lambda b: (b, 0, 0)),
            pl.BlockSpec((None, N, D), lambda b: (b, 0, 0)),
            pl.BlockSpec((None, N, D), lambda b: (b, 0, 0)),
        ],
        out_specs=pl.BlockSpec((None, N_TC, 3 * D), lambda b: (b, 0, 0)),
        out_shape=jax.ShapeDtypeStruct((B, N_TC, 3 * D), jnp.float32),
        compiler_params=pltpu.CompilerParams(
            dimension_semantics=("parallel",)),
    )(adj16[:, :N_TC, :], feats16, feats)


@functools.partial(
    pl.kernel,
    out_type=jax.ShapeDtypeStruct((B, R_SC, 3 * D), jnp.float32),
    mesh=plsc.VectorSubcoreMesh(core_axis_name="c", subcore_axis_name="s"),
    scratch_types=[
        pltpu.VMEM((N, D), jnp.float32),
        pltpu.VMEM((RW, N), jnp.float32),
        pltpu.VMEM((RW, 3 * D), jnp.float32),
    ],
)
def _sc_minmax(adj_hbm, feats_hbm, out_hbm, feats_v, adj_v, out_v):
    c = lax.axis_index("c")
    s = lax.axis_index("s")
    wid = s * 2 + c
    b = wid // NWB
    r0 = (N - R_SC) + (wid % NWB) * RW
    o0 = (wid % NWB) * RW
    pltpu.sync_copy(feats_hbm.at[b], feats_v)
    pltpu.sync_copy(adj_hbm.at[b, pl.ds(r0, RW)], adj_v)

    DH = DV // 2

    def pair_body(rp, _):
        for dh in range(2):
            def j16_body(jv, accs):
                mins0, maxs0, mins1, maxs1 = accs
                a0 = adj_v[2 * rp, pl.ds(jv * 16, 16)]
                a1 = adj_v[2 * rp + 1, pl.ds(jv * 16, 16)]
                mins0, maxs0 = list(mins0), list(maxs0)
                mins1, maxs1 = list(mins1), list(maxs1)
                for k in range(16):
                    j = jv * 16 + k
                    s0 = a0[k]
                    s1 = a1[k]
                    for dv in range(DH):
                        f = feats_v[j, pl.ds((dh * DH + dv) * 16, 16)]
                        p0 = s0 * f
                        p1 = s1 * f
                        mins0[dv] = jnp.minimum(mins0[dv], p0)
                        maxs0[dv] = jnp.maximum(maxs0[dv], p0)
                        mins1[dv] = jnp.minimum(mins1[dv], p1)
                        maxs1[dv] = jnp.maximum(maxs1[dv], p1)
                return (tuple(mins0), tuple(maxs0),
                        tuple(mins1), tuple(maxs1))

            pinf = tuple(jnp.full((16,), jnp.inf, jnp.float32)
                         for _ in range(DH))
            ninf = tuple(jnp.full((16,), -jnp.inf, jnp.float32)
                         for _ in range(DH))
            mins0, maxs0, mins1, maxs1 = lax.fori_loop(
                0, N // 16, j16_body, (pinf, ninf, pinf, ninf))
            for dv in range(DH):
                d0 = (dh * DH + dv) * 16
                sl = pl.ds(d0, 16)
                out_v[2 * rp, sl] = feats_v[r0 + 2 * rp, sl]
                out_v[2 * rp + 1, sl] = feats_v[r0 + 2 * rp + 1, sl]
                out_v[2 * rp, pl.ds(D + d0, 16)] = mins0[dv]
                out_v[2 * rp, pl.ds(2 * D + d0, 16)] = maxs0[dv]
                out_v[2 * rp + 1, pl.ds(D + d0, 16)] = mins1[dv]
                out_v[2 * rp + 1, pl.ds(2 * D + d0, 16)] = maxs1[dv]
        return 0

    lax.fori_loop(0, RW // 2, pair_body, 0)
    pltpu.sync_copy(out_v, out_hbm.at[b, pl.ds(o0, RW)])


def kernel(adjMs, feats):
    parts = []
    if N_TC > 0:
        adj16 = adjMs.astype(jnp.bfloat16)
        feats16 = feats.astype(jnp.bfloat16)
        parts.append(_tc_call(adj16, feats16, feats))
    if R_SC > 0:
        parts.append(_sc_minmax(adjMs, feats))
    out = parts[0] if len(parts) == 1 else jnp.concatenate(parts, axis=1)
    return (adjMs, out)


# speedup vs baseline: 7.4893x; 5.5230x over previous
import jax
import jax.numpy as jnp
from jax.experimental import pallas as pl
from jax.experimental.pallas import tpu as pltpu

B, N, D = 2, 512, 128
BI = 64


def _minmax_body(adj_ref, feats16_ref, feats32_ref, out_ref):
    adj = adj_ref[...]
    feats = feats16_ref[...]
    masked = adj[:, :, None] * feats[None, :, :]
    mn = masked
    mx = masked
    w = N
    while w > 16:
        half = w // 2
        mn = jnp.minimum(mn[:, :half], mn[:, half:w])
        mx = jnp.maximum(mx[:, :half], mx[:, half:w])
        w = half
    mins = jnp.min(mn, axis=1)
    maxs = jnp.max(mx, axis=1)
    i0 = pl.program_id(1) * BI
    out_ref[:, 0:D] = feats32_ref[pl.ds(i0, BI), :]
    out_ref[:, D:2 * D] = mins.astype(jnp.float32)
    out_ref[:, 2 * D:3 * D] = maxs.astype(jnp.float32)


def kernel(adjMs, feats):
    adj16 = adjMs.astype(jnp.bfloat16)
    feats16 = feats.astype(jnp.bfloat16)
    out = pl.pallas_call(
        _minmax_body,
        grid=(B, N // BI),
        in_specs=[
            pl.BlockSpec((None, BI, N), lambda b, i: (b, i, 0)),
            pl.BlockSpec((None, N, D), lambda b, i: (b, 0, 0)),
            pl.BlockSpec((None, N, D), lambda b, i: (b, 0, 0)),
        ],
        out_specs=pl.BlockSpec((None, BI, 3 * D), lambda b, i: (b, i, 0)),
        out_shape=jax.ShapeDtypeStruct((B, N, 3 * D), jnp.float32),
        compiler_params=pltpu.CompilerParams(
            dimension_semantics=("parallel", "parallel")),
    )(adj16, feats16, feats)
    return (adjMs, out)


# speedup vs baseline: 8.3216x; 1.1111x over previous
import jax
import jax.numpy as jnp
from jax.experimental import pallas as pl
from jax.experimental.pallas import tpu as pltpu

B, N, D = 2, 512, 128
BI = 512


def _minmax_body(adj_ref, feats_ref, out_ref):
    adj = adj_ref[...].astype(jnp.bfloat16)
    feats = feats_ref[...].astype(jnp.bfloat16)
    masked = adj[:, :, None] * feats[None, :, :]
    mn = masked
    mx = masked
    w = N
    while w > 16:
        half = w // 2
        mn = jnp.minimum(mn[:, :half], mn[:, half:w])
        mx = jnp.maximum(mx[:, :half], mx[:, half:w])
        w = half
    mins = jnp.min(mn, axis=1)
    maxs = jnp.max(mx, axis=1)
    i0 = pl.program_id(1) * BI
    out_ref[:, 0:D] = feats_ref[pl.ds(i0, BI), :]
    out_ref[:, D:2 * D] = mins.astype(jnp.float32)
    out_ref[:, 2 * D:3 * D] = maxs.astype(jnp.float32)


def kernel(adjMs, feats):
    out = pl.pallas_call(
        _minmax_body,
        grid=(B, N // BI),
        in_specs=[
            pl.BlockSpec((None, BI, N), lambda b, i: (b, i, 0)),
            pl.BlockSpec((None, N, D), lambda b, i: (b, 0, 0)),
        ],
        out_specs=pl.BlockSpec((None, BI, 3 * D), lambda b, i: (b, i, 0)),
        out_shape=jax.ShapeDtypeStruct((B, N, 3 * D), jnp.float32),
        compiler_params=pltpu.CompilerParams(
            dimension_semantics=("parallel", "parallel")),
    )(adjMs, feats)
    return (adjMs, out)
